# baseline (speedup 1.0000x reference)
import jax
import jax.numpy as jnp
from jax.experimental import pallas as pl
from jax.experimental.pallas import tpu as pltpu

_BI = 2000
_BK = 1024


def _sage_kernel(adj_ref, xc_ref, xr_ref, w_ref, b_ref, out_ref,
                 acc_ref, deg_ref, *, n, d_in):
    k = pl.program_id(1)
    nk = pl.num_programs(1)

    @pl.when(k == 0)
    def _init():
        acc_ref[...] = jnp.zeros_like(acc_ref)
        deg_ref[...] = jnp.zeros_like(deg_ref)

    a_raw = adj_ref[...]
    col = k * _BK + jax.lax.broadcasted_iota(jnp.int32, a_raw.shape, 1)
    m = (a_raw == 1) & (col < n)
    a = m.astype(jnp.bfloat16)
    acc_ref[...] += jnp.dot(a, xc_ref[...], preferred_element_type=jnp.float32)
    deg_ref[...] += jnp.sum(m, axis=1, keepdims=True).astype(jnp.float32)

    @pl.when(k == nk - 1)
    def _epilogue():
        x1 = acc_ref[...] / deg_ref[...]
        w = w_ref[...]
        out_ref[...] = (
            jnp.dot(x1, w[:d_in], preferred_element_type=jnp.float32)
            + jnp.dot(xr_ref[...], w[d_in:], preferred_element_type=jnp.float32)
            + b_ref[...]
        )


def kernel(x, adj, weight, bias):
    n, d_in = x.shape
    d_out = weight.shape[1]
    nk = pl.cdiv(n, _BK)
    pad = nk * _BK - n
    xc = jnp.concatenate([x, jnp.zeros((pad, d_in), x.dtype)], axis=0) if pad else x
    xc = xc.astype(jnp.bfloat16)
    bias2d = bias.reshape(1, d_out)

    grid = (n // _BI, nk)
    out = pl.pallas_call(
        lambda *refs: _sage_kernel(*refs, n=n, d_in=d_in),
        grid=grid,
        in_specs=[
            pl.BlockSpec((_BI, _BK), lambda i, k: (i, k)),
            pl.BlockSpec((_BK, d_in), lambda i, k: (k, 0)),
            pl.BlockSpec((_BI, d_in), lambda i, k: (i, 0)),
            pl.BlockSpec((2 * d_in, d_out), lambda i, k: (0, 0)),
            pl.BlockSpec((1, d_out), lambda i, k: (0, 0)),
        ],
        out_specs=pl.BlockSpec((_BI, d_out), lambda i, k: (i, 0)),
        out_shape=jax.ShapeDtypeStruct((n, d_out), jnp.float32),
        scratch_shapes=[
            pltpu.VMEM((_BI, d_out), jnp.float32),
            pltpu.VMEM((_BI, 1), jnp.float32),
        ],
        compiler_params=pltpu.CompilerParams(
            dimension_semantics=("parallel", "arbitrary"),
        ),
    )(adj, xc, x, weight, bias2d)
    return out


# speedup vs baseline: 1.0636x; 1.0636x over previous
import jax
import jax.numpy as jnp
from jax.experimental import pallas as pl
from jax.experimental.pallas import tpu as pltpu

_BI = 2000
_BK = 1024


def _sage_kernel(adj_ref, xc_ref, xr_ref, w_ref, b_ref, out_ref,
                 acc_ref, deg_ref, *, n, d_in):
    k = pl.program_id(1)
    nk = pl.num_programs(1)

    @pl.when(k == 0)
    def _init():
        acc_ref[...] = jnp.zeros_like(acc_ref)
        deg_ref[...] = jnp.zeros_like(deg_ref)

    a = adj_ref[...].astype(jnp.float32)
    acc_ref[...] += jnp.dot(a, xc_ref[...], preferred_element_type=jnp.float32)

    @pl.when(k < nk - 1)
    def _deg_plain():
        deg_ref[...] += jnp.sum(a, axis=1, keepdims=True)

    @pl.when(k == nk - 1)
    def _deg_tail_and_epilogue():
        col = k * _BK + jax.lax.broadcasted_iota(jnp.int32, a.shape, 1)
        deg_ref[...] += jnp.sum(jnp.where(col < n, a, 0.0), axis=1,
                                keepdims=True)
        x1 = acc_ref[...] / deg_ref[...]
        w = w_ref[...]
        out_ref[...] = (
            jnp.dot(x1, w[:d_in], preferred_element_type=jnp.float32)
            + jnp.dot(xr_ref[...], w[d_in:], preferred_element_type=jnp.float32)
            + b_ref[...]
        )


def kernel(x, adj, weight, bias):
    n, d_in = x.shape
    d_out = weight.shape[1]
    nk = pl.cdiv(n, _BK)
    pad = nk * _BK - n
    xc = jnp.concatenate([x, jnp.zeros((pad, d_in), x.dtype)], axis=0) if pad else x
    bias2d = bias.reshape(1, d_out)

    grid = (n // _BI, nk)
    out = pl.pallas_call(
        lambda *refs: _sage_kernel(*refs, n=n, d_in=d_in),
        grid=grid,
        in_specs=[
            pl.BlockSpec((_BI, _BK), lambda i, k: (i, k)),
            pl.BlockSpec((_BK, d_in), lambda i, k: (k, 0)),
            pl.BlockSpec((_BI, d_in), lambda i, k: (i, 0)),
            pl.BlockSpec((2 * d_in, d_out), lambda i, k: (0, 0)),
            pl.BlockSpec((1, d_out), lambda i, k: (0, 0)),
        ],
        out_specs=pl.BlockSpec((_BI, d_out), lambda i, k: (i, 0)),
        out_shape=jax.ShapeDtypeStruct((n, d_out), jnp.float32),
        scratch_shapes=[
            pltpu.VMEM((_BI, d_out), jnp.float32),
            pltpu.VMEM((_BI, 1), jnp.float32),
        ],
        compiler_params=pltpu.CompilerParams(
            dimension_semantics=("parallel", "arbitrary"),
        ),
    )(adj, xc, x, weight, bias2d)
    return out
